# baseline (speedup 1.0000x reference)
import functools

import jax
import jax.numpy as jnp
from jax import lax
from jax.experimental import pallas as pl
from jax.experimental.pallas import tpu as pltpu
from jax.experimental.pallas import tpu_sc as plsc

_N = 10000
_E = 320000
_NG = 16
_D = 128
_FC = 128
_NC = 10

_NP = 10240
_EP = 327680
_B = 128
_K = 2
_NW = 32
_NBLK = (_EP // _NW) // _B
_NCHUNK = _NBLK // _K
_RPT = _NP // 16
_R = 1280
_G = _NP // _R

_f32 = jnp.float32


_C = 4
_NCH = _NBLK // _C

_NBLK0 = 160
_NCH0 = _NBLK0 // _C


def _prop_body(g_hbm, src_hbm, dst_hbm, out_hbm, sidx, didx, rows, acc,
               isem, gsem, ssem0, ssem1):
    c = lax.axis_index("c")
    s = lax.axis_index("s")
    ssems = (ssem0, ssem1)

    @pl.when(c == 0)
    def _core0():
        def zrow(i, _):
            for j in range(_D // 16):
                rows[0, i, pl.ds(j * 16, 16)] = jnp.zeros((16,), _f32)
            return 0
        lax.fori_loop(0, _B, zrow, 0)
        rbase = s * _RPT
        for r in range(_RPT // _B):
            pltpu.sync_copy(rows.at[0], acc.at[pl.ds(rbase + r * _B, _B)])
        plsc.subcore_barrier()

        row0 = s * _NBLK0

        def ifetch(ci):
            h = lax.rem(ci, 2) * _C
            pltpu.async_copy(src_hbm.at[pl.ds(row0 + ci * _C, _C)],
                             sidx.at[pl.ds(h, _C)], isem)
            pltpu.async_copy(dst_hbm.at[pl.ds(row0 + ci * _C, _C)],
                             didx.at[pl.ds(h, _C)], isem)

        def iwait():
            for _ in range(2):
                pltpu.make_async_copy(src_hbm.at[pl.ds(0, _C)],
                                      sidx.at[pl.ds(0, _C)], isem).wait()

        def swait(p):
            pltpu.make_async_copy(rows.at[0], acc.at[pl.ds(0, _B)],
                                  ssems[p]).wait()

        ifetch(0)

        def chunk(ci, _):
            h = lax.rem(ci, 2) * _C
            iwait()
            pltpu.async_copy(g_hbm.at[sidx.at[h]], rows.at[0], gsem)

            @pl.when(ci > 0)
            def _():
                swait(1)
            ifetch(lax.rem(ci + 1, _NCH0))

            for j in range(_C):
                pltpu.make_async_copy(g_hbm.at[sidx.at[h]], rows.at[j % 2],
                                      gsem).wait()
                pltpu.async_copy(rows.at[j % 2], acc.at[didx.at[h + j]],
                                 ssems[j % 2], add=True)
                if j > 0:
                    swait((j - 1) % 2)
                if j < _C - 1:
                    pltpu.async_copy(g_hbm.at[sidx.at[h + j + 1]],
                                     rows.at[(j + 1) % 2], gsem)
            return 0
        lax.fori_loop(0, _NCH0, chunk, 0)

        swait(1)
        iwait()
        plsc.subcore_barrier()
        pltpu.sync_copy(acc.at[pl.ds(rbase, _RPT)],
                        out_hbm.at[pl.ds(rbase, _RPT)])


@functools.cache
def _prop_kernel():
    mesh = plsc.VectorSubcoreMesh(core_axis_name="c", subcore_axis_name="s")
    return pl.kernel(
        _prop_body, mesh=mesh,
        out_type=jax.ShapeDtypeStruct((_NP, _D), _f32),
        scratch_types=[
            pltpu.VMEM((2 * _C, _B), jnp.int32),
            pltpu.VMEM((2 * _C, _B), jnp.int32),
            pltpu.VMEM((2, _B, _D), _f32),
            pltpu.VMEM_SHARED((_NP, _D), _f32),
            pltpu.SemaphoreType.DMA,
            pltpu.SemaphoreType.DMA,
            pltpu.SemaphoreType.DMA,
            pltpu.SemaphoreType.DMA,
        ],
    )


def _deg_body(dst_hbm, out_hbm, didx, buf, acc, isem, ssem):
    c = lax.axis_index("c")
    s = lax.axis_index("s")
    wid = s * 2 + c

    def zrow(i, _):
        for j in range(_D // 16):
            buf[i, pl.ds(j * 16, 16)] = jnp.zeros((16,), _f32)
        return 0
    lax.fori_loop(0, _B, zrow, 0)
    rbase = s * _RPT
    for r in range(_RPT // _B):
        pltpu.sync_copy(buf, acc.at[pl.ds(rbase + r * _B, _B)])

    def orow(i, _):
        for j in range(_D // 16):
            buf[i, pl.ds(j * 16, 16)] = jnp.ones((16,), _f32)
        return 0
    lax.fori_loop(0, _B, orow, 0)
    plsc.subcore_barrier()

    row0 = wid * _NBLK

    def ifetch(ci):
        h = lax.rem(ci, 2) * _C
        pltpu.async_copy(dst_hbm.at[pl.ds(row0 + ci * _C, _C)],
                         didx.at[pl.ds(h, _C)], isem)

    def iwait():
        pltpu.make_async_copy(dst_hbm.at[pl.ds(0, _C)],
                              didx.at[pl.ds(0, _C)], isem).wait()

    def swait():
        pltpu.make_async_copy(buf, acc.at[pl.ds(0, _B)], ssem).wait()

    ifetch(0)

    def chunk(ci, _):
        h = lax.rem(ci, 2) * _C
        iwait()
        @pl.when(ci > 0)
        def _():
            for _k in range(_C):
                swait()
        ifetch(lax.rem(ci + 1, _NCH))
        for j in range(_C):
            pltpu.async_copy(buf, acc.at[didx.at[h + j]], ssem, add=True)
        return 0
    lax.fori_loop(0, _NCH, chunk, 0)

    for _k in range(_C):
        swait()
    iwait()
    plsc.subcore_barrier()
    pltpu.sync_copy(acc.at[pl.ds(rbase, _RPT)],
                    out_hbm.at[c, pl.ds(rbase, _RPT)])


@functools.cache
def _deg_kernel():
    mesh = plsc.VectorSubcoreMesh(core_axis_name="c", subcore_axis_name="s")
    return pl.kernel(
        _deg_body, mesh=mesh,
        out_type=jax.ShapeDtypeStruct((2, _NP, _D), _f32),
        scratch_types=[
            pltpu.VMEM((2 * _C, _B), jnp.int32),
            pltpu.VMEM((_B, _D), _f32),
            pltpu.VMEM_SHARED((_NP, _D), _f32),
            pltpu.SemaphoreType.DMA,
            pltpu.SemaphoreType.DMA,
        ],
    )


def _k1_body(degp_ref, x_ref, dinv_ref, t0_ref):
    i = pl.program_id(0)
    deg = degp_ref[0][:, 0:1] + degp_ref[1][:, 0:1] + 1.0
    d = lax.rsqrt(deg)
    rid = lax.broadcasted_iota(jnp.int32, (_R, 1), 0) + i * _R
    d = jnp.where(rid < _N, d, 0.0)
    dinv_ref[...] = d
    t0_ref[...] = d * x_ref[...]


def _k2_body(p_ref, u_ref, dinv_ref, o_ref):
    d = dinv_ref[...]
    o_ref[...] = (d * d) * (p_ref[...] + u_ref[...])


def _k3_body(p_ref, u_ref, dinv_ref, w_ref, b_ref, o_ref):
    d = dinv_ref[...]
    h = d * (p_ref[...] + u_ref[...])
    a = jnp.dot(h, w_ref[...], preferred_element_type=_f32) + b_ref[...]
    o_ref[...] = d * jnp.maximum(a, 0.0)


def _k5_body(p_ref, u_ref, dinv_ref, w2_ref, b2_ref, batch_ref,
             f1w_ref, f1b_ref, f2w_ref, f2b_ref, out_ref, sums, counts):
    i = pl.program_id(0)

    @pl.when(i == 0)
    def _():
        sums[...] = jnp.zeros_like(sums)
        counts[...] = jnp.zeros_like(counts)

    d = dinv_ref[...]
    h = d * (p_ref[...] + u_ref[...])
    a = jnp.dot(h, w2_ref[...], preferred_element_type=_f32) + b2_ref[...]
    a = jnp.maximum(a, 0.0)
    onehot = (batch_ref[...] ==
              lax.broadcasted_iota(jnp.int32, (1, _NG), 1)).astype(_f32)
    sums[...] += lax.dot_general(onehot, a, (((0,), (0,)), ((), ())),
                                 preferred_element_type=_f32)
    counts[...] += lax.dot_general(onehot, jnp.ones((_R, _D), _f32),
                                   (((0,), (0,)), ((), ())),
                                   preferred_element_type=_f32)

    @pl.when(i == _G - 1)
    def _():
        g = sums[...] / jnp.maximum(counts[...], 1.0)
        z = jnp.dot(g, f1w_ref[...], preferred_element_type=_f32) + f1b_ref[...]
        z = jnp.maximum(z, 0.0)
        logits = jnp.dot(z, f2w_ref[...], preferred_element_type=_f32) + f2b_ref[...]
        m = jnp.max(logits, axis=-1, keepdims=True)
        e = jnp.exp(logits - m)
        out_ref[...] = e / jnp.sum(e, axis=-1, keepdims=True)


def _nd(block, im):
    return pl.BlockSpec(block, im)


_full2 = lambda shape: pl.BlockSpec(shape, lambda i: (0, 0))
_rows = lambda w: pl.BlockSpec((_R, w), lambda i: (i, 0))
_parts = pl.BlockSpec((2, _R, _D), lambda i: (0, i, 0))


def _k1_call(degp, x_pad):
    return pl.pallas_call(
        _k1_body,
        grid=(_G,),
        in_specs=[_parts, _rows(_D)],
        out_specs=[_rows(1), _rows(_D)],
        out_shape=[jax.ShapeDtypeStruct((_NP, 1), _f32),
                   jax.ShapeDtypeStruct((_NP, _D), _f32)],
    )(degp, x_pad)


def _k2_call(p, u, dinv):
    return pl.pallas_call(
        _k2_body,
        grid=(_G,),
        in_specs=[_rows(_D), _rows(_D), _rows(1)],
        out_specs=_rows(_D),
        out_shape=jax.ShapeDtypeStruct((_NP, _D), _f32),
    )(p, u, dinv)


def _k3_call(p, u, dinv, W, b):
    return pl.pallas_call(
        _k3_body,
        grid=(_G,),
        in_specs=[_rows(_D), _rows(_D), _rows(1), _full2((_D, _D)), _full2((1, _D))],
        out_specs=_rows(_D),
        out_shape=jax.ShapeDtypeStruct((_NP, _D), _f32),
    )(p, u, dinv, W, b)


def _k5_call(p, u, dinv, W2, b2, batch2d, f1w, f1b, f2w, f2b):
    return pl.pallas_call(
        _k5_body,
        grid=(_G,),
        in_specs=[_rows(_D), _rows(_D), _rows(1), _full2((_D, _D)), _full2((1, _D)),
                  _rows(1), _full2((_D, _FC)), _full2((1, _FC)),
                  _full2((_FC, _NC)), _full2((1, _NC))],
        out_specs=_full2((_NG, _NC)),
        out_shape=jax.ShapeDtypeStruct((_NG, _NC), _f32),
        scratch_shapes=[pltpu.VMEM((_NG, _D), _f32),
                        pltpu.VMEM((_NG, _D), _f32)],
    )(p, u, dinv, W2, b2, batch2d, f1w, f1b, f2w, f2b)


def kernel(x, edge_index, batch, W1, b1, W2, b2, fc1_w, fc1_b, fc2_w, fc2_b):
    src = edge_index[0]
    dst = edge_index[1]
    pad_e = jnp.full((_EP - _E,), _N, jnp.int32)
    src2d = jnp.concatenate([src, pad_e]).reshape(_EP // _B, _B)
    dst2d = jnp.concatenate([dst, pad_e]).reshape(_EP // _B, _B)
    x_pad = jnp.concatenate([x, jnp.zeros((_NP - _N, _D), _f32)], axis=0)
    batch2d = jnp.concatenate(
        [batch, jnp.full((_NP - _N,), _NG, jnp.int32)]).reshape(_NP, 1)

    deg = _deg_kernel()
    prop = _prop_kernel()

    degp = deg(dst2d)
    dinv, t0 = _k1_call(degp, x_pad)
    p = prop(t0, src2d, dst2d)
    t1 = _k2_call(p, t0, dinv)
    p = prop(t1, src2d, dst2d)
    g2 = _k3_call(p, t1, dinv, W1, b1.reshape(1, _D))
    p = prop(g2, src2d, dst2d)
    t3 = _k2_call(p, g2, dinv)
    p = prop(t3, src2d, dst2d)
    out = _k5_call(p, t3, dinv, W2, b2.reshape(1, _D), batch2d,
                   fc1_w, fc1_b.reshape(1, _FC), fc2_w, fc2_b.reshape(1, _NC))
    return out


# speedup vs baseline: 1.5215x; 1.5215x over previous
import functools

import jax
import jax.numpy as jnp
from jax import lax
from jax.experimental import pallas as pl
from jax.experimental.pallas import tpu as pltpu
from jax.experimental.pallas import tpu_sc as plsc

_N = 10000
_E = 320000
_NG = 16
_D = 128
_FC = 128
_NC = 10

_NP = 10240
_EP = 327680
_B = 128
_K = 2
_NW = 32
_NBLK = (_EP // _NW) // _B
_NCHUNK = _NBLK // _K
_RPT = _NP // 16
_R = 1280
_G = _NP // _R

_f32 = jnp.float32


_C = 4
_NCH = _NBLK // _C

_NBLK0 = 144
_NBLK1 = 16
_NCH0 = _NBLK0 // _C
_NCH1 = _NBLK1 // _C


def _prop_body(g_hbm, src_hbm, dst_hbm, out_hbm, sidx, didx, rows, acc,
               isem, gsem, ssem0, ssem1):
    c = lax.axis_index("c")
    s = lax.axis_index("s")
    ssems = (ssem0, ssem1)

    def zrow(i, _):
        for j in range(_D // 16):
            rows[0, i, pl.ds(j * 16, 16)] = jnp.zeros((16,), _f32)
        return 0
    lax.fori_loop(0, _B, zrow, 0)
    rbase = s * _RPT
    for r in range(_RPT // _B):
        pltpu.sync_copy(rows.at[0], acc.at[pl.ds(rbase + r * _B, _B)])
    plsc.subcore_barrier()

    row0 = lax.select(c == 0, s * _NBLK0, 16 * _NBLK0 + s * _NBLK1)
    nch = lax.select(c == 0, _NCH0, _NCH1)

    def ifetch(ci):
        h = lax.rem(ci, 2) * _C
        pltpu.async_copy(src_hbm.at[pl.ds(row0 + ci * _C, _C)],
                         sidx.at[pl.ds(h, _C)], isem)
        pltpu.async_copy(dst_hbm.at[pl.ds(row0 + ci * _C, _C)],
                         didx.at[pl.ds(h, _C)], isem)

    def iwait():
        for _ in range(2):
            pltpu.make_async_copy(src_hbm.at[pl.ds(0, _C)],
                                  sidx.at[pl.ds(0, _C)], isem).wait()

    def swait(p):
        pltpu.make_async_copy(rows.at[0], acc.at[pl.ds(0, _B)],
                              ssems[p]).wait()

    ifetch(0)

    def chunk(ci, _):
        h = lax.rem(ci, 2) * _C
        iwait()
        pltpu.async_copy(g_hbm.at[sidx.at[h]], rows.at[0], gsem)

        @pl.when(ci > 0)
        def _():
            swait(1)
        ifetch(lax.rem(ci + 1, nch))

        for j in range(_C):
            pltpu.make_async_copy(g_hbm.at[sidx.at[h]], rows.at[j % 2],
                                  gsem).wait()
            pltpu.async_copy(rows.at[j % 2], acc.at[didx.at[h + j]],
                             ssems[j % 2], add=True)
            if j > 0:
                swait((j - 1) % 2)
            if j < _C - 1:
                pltpu.async_copy(g_hbm.at[sidx.at[h + j + 1]],
                                 rows.at[(j + 1) % 2], gsem)
        return 0
    lax.fori_loop(0, nch, chunk, 0)

    swait(1)
    iwait()
    plsc.subcore_barrier()
    pltpu.sync_copy(acc.at[pl.ds(rbase, _RPT)],
                    out_hbm.at[c, pl.ds(rbase, _RPT)])


@functools.cache
def _prop_kernel():
    mesh = plsc.VectorSubcoreMesh(core_axis_name="c", subcore_axis_name="s")
    return pl.kernel(
        _prop_body, mesh=mesh,
        out_type=jax.ShapeDtypeStruct((2, _NP, _D), _f32),
        scratch_types=[
            pltpu.VMEM((2 * _C, _B), jnp.int32),
            pltpu.VMEM((2 * _C, _B), jnp.int32),
            pltpu.VMEM((2, _B, _D), _f32),
            pltpu.VMEM_SHARED((_NP, _D), _f32),
            pltpu.SemaphoreType.DMA,
            pltpu.SemaphoreType.DMA,
            pltpu.SemaphoreType.DMA,
            pltpu.SemaphoreType.DMA,
        ],
    )


def _deg_body(dst_hbm, out_hbm, didx, buf, acc, isem, ssem):
    c = lax.axis_index("c")
    s = lax.axis_index("s")
    wid = s * 2 + c

    def zrow(i, _):
        for j in range(_D // 16):
            buf[i, pl.ds(j * 16, 16)] = jnp.zeros((16,), _f32)
        return 0
    lax.fori_loop(0, _B, zrow, 0)
    rbase = s * _RPT
    for r in range(_RPT // _B):
        pltpu.sync_copy(buf, acc.at[pl.ds(rbase + r * _B, _B)])

    def orow(i, _):
        for j in range(_D // 16):
            buf[i, pl.ds(j * 16, 16)] = jnp.ones((16,), _f32)
        return 0
    lax.fori_loop(0, _B, orow, 0)
    plsc.subcore_barrier()

    row0 = wid * _NBLK

    def ifetch(ci):
        h = lax.rem(ci, 2) * _C
        pltpu.async_copy(dst_hbm.at[pl.ds(row0 + ci * _C, _C)],
                         didx.at[pl.ds(h, _C)], isem)

    def iwait():
        pltpu.make_async_copy(dst_hbm.at[pl.ds(0, _C)],
                              didx.at[pl.ds(0, _C)], isem).wait()

    def swait():
        pltpu.make_async_copy(buf, acc.at[pl.ds(0, _B)], ssem).wait()

    ifetch(0)

    def chunk(ci, _):
        h = lax.rem(ci, 2) * _C
        iwait()
        @pl.when(ci > 0)
        def _():
            for _k in range(_C):
                swait()
        ifetch(lax.rem(ci + 1, _NCH))
        for j in range(_C):
            pltpu.async_copy(buf, acc.at[didx.at[h + j]], ssem, add=True)
        return 0
    lax.fori_loop(0, _NCH, chunk, 0)

    for _k in range(_C):
        swait()
    iwait()
    plsc.subcore_barrier()
    pltpu.sync_copy(acc.at[pl.ds(rbase, _RPT)],
                    out_hbm.at[c, pl.ds(rbase, _RPT)])


@functools.cache
def _deg_kernel():
    mesh = plsc.VectorSubcoreMesh(core_axis_name="c", subcore_axis_name="s")
    return pl.kernel(
        _deg_body, mesh=mesh,
        out_type=jax.ShapeDtypeStruct((2, _NP, _D), _f32),
        scratch_types=[
            pltpu.VMEM((2 * _C, _B), jnp.int32),
            pltpu.VMEM((_B, _D), _f32),
            pltpu.VMEM_SHARED((_NP, _D), _f32),
            pltpu.SemaphoreType.DMA,
            pltpu.SemaphoreType.DMA,
        ],
    )


def _k1_body(degp_ref, x_ref, dinv_ref, t0_ref):
    i = pl.program_id(0)
    deg = degp_ref[0][:, 0:1] + degp_ref[1][:, 0:1] + 1.0
    d = lax.rsqrt(deg)
    rid = lax.broadcasted_iota(jnp.int32, (_R, 1), 0) + i * _R
    d = jnp.where(rid < _N, d, 0.0)
    dinv_ref[...] = d
    t0_ref[...] = d * x_ref[...]


def _k2_body(p_ref, u_ref, dinv_ref, o_ref):
    d = dinv_ref[...]
    o_ref[...] = (d * d) * (p_ref[0] + p_ref[1] + u_ref[...])


def _k3_body(p_ref, u_ref, dinv_ref, w_ref, b_ref, o_ref):
    d = dinv_ref[...]
    h = d * (p_ref[0] + p_ref[1] + u_ref[...])
    a = jnp.dot(h, w_ref[...], preferred_element_type=_f32) + b_ref[...]
    o_ref[...] = d * jnp.maximum(a, 0.0)


def _k5_body(p_ref, u_ref, dinv_ref, w2_ref, b2_ref, batch_ref,
             f1w_ref, f1b_ref, f2w_ref, f2b_ref, out_ref, sums, counts):
    i = pl.program_id(0)

    @pl.when(i == 0)
    def _():
        sums[...] = jnp.zeros_like(sums)
        counts[...] = jnp.zeros_like(counts)

    d = dinv_ref[...]
    h = d * (p_ref[0] + p_ref[1] + u_ref[...])
    a = jnp.dot(h, w2_ref[...], preferred_element_type=_f32) + b2_ref[...]
    a = jnp.maximum(a, 0.0)
    onehot = (batch_ref[...] ==
              lax.broadcasted_iota(jnp.int32, (1, _NG), 1)).astype(_f32)
    sums[...] += lax.dot_general(onehot, a, (((0,), (0,)), ((), ())),
                                 preferred_element_type=_f32)
    counts[...] += lax.dot_general(onehot, jnp.ones((_R, _D), _f32),
                                   (((0,), (0,)), ((), ())),
                                   preferred_element_type=_f32)

    @pl.when(i == _G - 1)
    def _():
        g = sums[...] / jnp.maximum(counts[...], 1.0)
        z = jnp.dot(g, f1w_ref[...], preferred_element_type=_f32) + f1b_ref[...]
        z = jnp.maximum(z, 0.0)
        logits = jnp.dot(z, f2w_ref[...], preferred_element_type=_f32) + f2b_ref[...]
        m = jnp.max(logits, axis=-1, keepdims=True)
        e = jnp.exp(logits - m)
        out_ref[...] = e / jnp.sum(e, axis=-1, keepdims=True)


def _nd(block, im):
    return pl.BlockSpec(block, im)


_full2 = lambda shape: pl.BlockSpec(shape, lambda i: (0, 0))
_rows = lambda w: pl.BlockSpec((_R, w), lambda i: (i, 0))
_parts = pl.BlockSpec((2, _R, _D), lambda i: (0, i, 0))


def _k1_call(degp, x_pad):
    return pl.pallas_call(
        _k1_body,
        grid=(_G,),
        in_specs=[_parts, _rows(_D)],
        out_specs=[_rows(1), _rows(_D)],
        out_shape=[jax.ShapeDtypeStruct((_NP, 1), _f32),
                   jax.ShapeDtypeStruct((_NP, _D), _f32)],
    )(degp, x_pad)


def _k2_call(p, u, dinv):
    return pl.pallas_call(
        _k2_body,
        grid=(_G,),
        in_specs=[_parts, _rows(_D), _rows(1)],
        out_specs=_rows(_D),
        out_shape=jax.ShapeDtypeStruct((_NP, _D), _f32),
    )(p, u, dinv)


def _k3_call(p, u, dinv, W, b):
    return pl.pallas_call(
        _k3_body,
        grid=(_G,),
        in_specs=[_parts, _rows(_D), _rows(1), _full2((_D, _D)), _full2((1, _D))],
        out_specs=_rows(_D),
        out_shape=jax.ShapeDtypeStruct((_NP, _D), _f32),
    )(p, u, dinv, W, b)


def _k5_call(p, u, dinv, W2, b2, batch2d, f1w, f1b, f2w, f2b):
    return pl.pallas_call(
        _k5_body,
        grid=(_G,),
        in_specs=[_parts, _rows(_D), _rows(1), _full2((_D, _D)), _full2((1, _D)),
                  _rows(1), _full2((_D, _FC)), _full2((1, _FC)),
                  _full2((_FC, _NC)), _full2((1, _NC))],
        out_specs=_full2((_NG, _NC)),
        out_shape=jax.ShapeDtypeStruct((_NG, _NC), _f32),
        scratch_shapes=[pltpu.VMEM((_NG, _D), _f32),
                        pltpu.VMEM((_NG, _D), _f32)],
    )(p, u, dinv, W2, b2, batch2d, f1w, f1b, f2w, f2b)


def kernel(x, edge_index, batch, W1, b1, W2, b2, fc1_w, fc1_b, fc2_w, fc2_b):
    src = edge_index[0]
    dst = edge_index[1]
    pad_e = jnp.full((_EP - _E,), _N, jnp.int32)
    src2d = jnp.concatenate([src, pad_e]).reshape(_EP // _B, _B)
    dst2d = jnp.concatenate([dst, pad_e]).reshape(_EP // _B, _B)
    x_pad = jnp.concatenate([x, jnp.zeros((_NP - _N, _D), _f32)], axis=0)
    batch2d = jnp.concatenate(
        [batch, jnp.full((_NP - _N,), _NG, jnp.int32)]).reshape(_NP, 1)

    deg = _deg_kernel()
    prop = _prop_kernel()

    degp = deg(dst2d)
    dinv, t0 = _k1_call(degp, x_pad)
    p = prop(t0, src2d, dst2d)
    t1 = _k2_call(p, t0, dinv)
    p = prop(t1, src2d, dst2d)
    g2 = _k3_call(p, t1, dinv, W1, b1.reshape(1, _D))
    p = prop(g2, src2d, dst2d)
    t3 = _k2_call(p, g2, dinv)
    p = prop(t3, src2d, dst2d)
    out = _k5_call(p, t3, dinv, W2, b2.reshape(1, _D), batch2d,
                   fc1_w, fc1_b.reshape(1, _FC), fc2_w, fc2_b.reshape(1, _NC))
    return out


# speedup vs baseline: 1.5231x; 1.0011x over previous
import functools

import jax
import jax.numpy as jnp
from jax import lax
from jax.experimental import pallas as pl
from jax.experimental.pallas import tpu as pltpu
from jax.experimental.pallas import tpu_sc as plsc

_N = 10000
_E = 320000
_NG = 16
_D = 128
_FC = 128
_NC = 10

_NP = 10240
_EP = 327680
_B = 128
_K = 2
_NW = 32
_NBLK = (_EP // _NW) // _B
_NCHUNK = _NBLK // _K
_RPT = _NP // 16
_R = 1280
_G = _NP // _R

_f32 = jnp.float32


_C = 4
_NCH = _NBLK // _C

_NBLK0 = 144
_NBLK1 = 16
_NCH0 = _NBLK0 // _C
_NCH1 = _NBLK1 // _C


def _prop_body(g0_hbm, g1_hbm, src_hbm, dst_hbm, out_hbm, sidx, didx, rows,
               acc, isem, gsem, ssem0, ssem1):
    c = lax.axis_index("c")
    s = lax.axis_index("s")
    ssems = (ssem0, ssem1)

    def zrow(i, _):
        for j in range(_D // 16):
            rows[0, i, pl.ds(j * 16, 16)] = jnp.zeros((16,), _f32)
        return 0
    lax.fori_loop(0, _B, zrow, 0)
    rbase = s * _RPT
    for r in range(_RPT // _B):
        pltpu.sync_copy(rows.at[0], acc.at[pl.ds(rbase + r * _B, _B)])
    plsc.subcore_barrier()

    def run(g_hbm, row0, nch):
        def ifetch(ci):
            h = lax.rem(ci, 2) * _C
            pltpu.async_copy(src_hbm.at[pl.ds(row0 + ci * _C, _C)],
                             sidx.at[pl.ds(h, _C)], isem)
            pltpu.async_copy(dst_hbm.at[pl.ds(row0 + ci * _C, _C)],
                             didx.at[pl.ds(h, _C)], isem)

        def iwait():
            for _ in range(2):
                pltpu.make_async_copy(src_hbm.at[pl.ds(0, _C)],
                                      sidx.at[pl.ds(0, _C)], isem).wait()

        def swait(p):
            pltpu.make_async_copy(rows.at[0], acc.at[pl.ds(0, _B)],
                                  ssems[p]).wait()

        ifetch(0)

        def chunk(ci, _):
            h = lax.rem(ci, 2) * _C
            iwait()
            pltpu.async_copy(g_hbm.at[sidx.at[h]], rows.at[0], gsem)

            @pl.when(ci > 0)
            def _():
                swait(1)
            ifetch(lax.rem(ci + 1, nch))

            for j in range(_C):
                pltpu.make_async_copy(g_hbm.at[sidx.at[h]], rows.at[j % 2],
                                      gsem).wait()
                pltpu.async_copy(rows.at[j % 2], acc.at[didx.at[h + j]],
                                 ssems[j % 2], add=True)
                if j > 0:
                    swait((j - 1) % 2)
                if j < _C - 1:
                    pltpu.async_copy(g_hbm.at[sidx.at[h + j + 1]],
                                     rows.at[(j + 1) % 2], gsem)
            return 0
        lax.fori_loop(0, nch, chunk, 0)

        swait(1)
        iwait()

    @pl.when(c == 0)
    def _():
        run(g0_hbm, s * _NBLK0, _NCH0)

    @pl.when(c == 1)
    def _():
        run(g1_hbm, 16 * _NBLK0 + s * _NBLK1, _NCH1)

    plsc.subcore_barrier()
    pltpu.sync_copy(acc.at[pl.ds(rbase, _RPT)],
                    out_hbm.at[c, pl.ds(rbase, _RPT)])


@functools.cache
def _prop_kernel():
    mesh = plsc.VectorSubcoreMesh(core_axis_name="c", subcore_axis_name="s")
    return pl.kernel(
        _prop_body, mesh=mesh,
        out_type=jax.ShapeDtypeStruct((2, _NP, _D), _f32),
        scratch_types=[
            pltpu.VMEM((2 * _C, _B), jnp.int32),
            pltpu.VMEM((2 * _C, _B), jnp.int32),
            pltpu.VMEM((2, _B, _D), _f32),
            pltpu.VMEM_SHARED((_NP, _D), _f32),
            pltpu.SemaphoreType.DMA,
            pltpu.SemaphoreType.DMA,
            pltpu.SemaphoreType.DMA,
            pltpu.SemaphoreType.DMA,
        ],
    )


def _deg_body(dst_hbm, out_hbm, didx, buf, acc, isem, ssem):
    c = lax.axis_index("c")
    s = lax.axis_index("s")
    wid = s * 2 + c

    def zrow(i, _):
        for j in range(_D // 16):
            buf[i, pl.ds(j * 16, 16)] = jnp.zeros((16,), _f32)
        return 0
    lax.fori_loop(0, _B, zrow, 0)
    rbase = s * _RPT
    for r in range(_RPT // _B):
        pltpu.sync_copy(buf, acc.at[pl.ds(rbase + r * _B, _B)])

    def orow(i, _):
        for j in range(_D // 16):
            buf[i, pl.ds(j * 16, 16)] = jnp.ones((16,), _f32)
        return 0
    lax.fori_loop(0, _B, orow, 0)
    plsc.subcore_barrier()

    row0 = wid * _NBLK

    def ifetch(ci):
        h = lax.rem(ci, 2) * _C
        pltpu.async_copy(dst_hbm.at[pl.ds(row0 + ci * _C, _C)],
                         didx.at[pl.ds(h, _C)], isem)

    def iwait():
        pltpu.make_async_copy(dst_hbm.at[pl.ds(0, _C)],
                              didx.at[pl.ds(0, _C)], isem).wait()

    def swait():
        pltpu.make_async_copy(buf, acc.at[pl.ds(0, _B)], ssem).wait()

    ifetch(0)

    def chunk(ci, _):
        h = lax.rem(ci, 2) * _C
        iwait()
        @pl.when(ci > 0)
        def _():
            for _k in range(_C):
                swait()
        ifetch(lax.rem(ci + 1, _NCH))
        for j in range(_C):
            pltpu.async_copy(buf, acc.at[didx.at[h + j]], ssem, add=True)
        return 0
    lax.fori_loop(0, _NCH, chunk, 0)

    for _k in range(_C):
        swait()
    iwait()
    plsc.subcore_barrier()
    pltpu.sync_copy(acc.at[pl.ds(rbase, _RPT)],
                    out_hbm.at[c, pl.ds(rbase, _RPT)])


@functools.cache
def _deg_kernel():
    mesh = plsc.VectorSubcoreMesh(core_axis_name="c", subcore_axis_name="s")
    return pl.kernel(
        _deg_body, mesh=mesh,
        out_type=jax.ShapeDtypeStruct((2, _NP, _D), _f32),
        scratch_types=[
            pltpu.VMEM((2 * _C, _B), jnp.int32),
            pltpu.VMEM((_B, _D), _f32),
            pltpu.VMEM_SHARED((_NP, _D), _f32),
            pltpu.SemaphoreType.DMA,
            pltpu.SemaphoreType.DMA,
        ],
    )


def _k1_body(degp_ref, x_ref, dinv_ref, t0_ref, t0b_ref):
    i = pl.program_id(0)
    deg = degp_ref[0][:, 0:1] + degp_ref[1][:, 0:1] + 1.0
    d = lax.rsqrt(deg)
    rid = lax.broadcasted_iota(jnp.int32, (_R, 1), 0) + i * _R
    d = jnp.where(rid < _N, d, 0.0)
    dinv_ref[...] = d
    t0 = d * x_ref[...]
    t0_ref[...] = t0
    t0b_ref[...] = t0


def _k2_body(p_ref, u_ref, dinv_ref, o_ref, ob_ref):
    d = dinv_ref[...]
    o = (d * d) * (p_ref[0] + p_ref[1] + u_ref[...])
    o_ref[...] = o
    ob_ref[...] = o


def _k3_body(p_ref, u_ref, dinv_ref, w_ref, b_ref, o_ref, ob_ref):
    d = dinv_ref[...]
    h = d * (p_ref[0] + p_ref[1] + u_ref[...])
    a = jnp.dot(h, w_ref[...], preferred_element_type=_f32) + b_ref[...]
    o = d * jnp.maximum(a, 0.0)
    o_ref[...] = o
    ob_ref[...] = o


def _k5_body(p_ref, u_ref, dinv_ref, w2_ref, b2_ref, batch_ref,
             f1w_ref, f1b_ref, f2w_ref, f2b_ref, out_ref, sums, counts):
    i = pl.program_id(0)

    @pl.when(i == 0)
    def _():
        sums[...] = jnp.zeros_like(sums)
        counts[...] = jnp.zeros_like(counts)

    d = dinv_ref[...]
    h = d * (p_ref[0] + p_ref[1] + u_ref[...])
    a = jnp.dot(h, w2_ref[...], preferred_element_type=_f32) + b2_ref[...]
    a = jnp.maximum(a, 0.0)
    onehot = (batch_ref[...] ==
              lax.broadcasted_iota(jnp.int32, (1, _NG), 1)).astype(_f32)
    sums[...] += lax.dot_general(onehot, a, (((0,), (0,)), ((), ())),
                                 preferred_element_type=_f32)
    counts[...] += lax.dot_general(onehot, jnp.ones((_R, _D), _f32),
                                   (((0,), (0,)), ((), ())),
                                   preferred_element_type=_f32)

    @pl.when(i == _G - 1)
    def _():
        g = sums[...] / jnp.maximum(counts[...], 1.0)
        z = jnp.dot(g, f1w_ref[...], preferred_element_type=_f32) + f1b_ref[...]
        z = jnp.maximum(z, 0.0)
        logits = jnp.dot(z, f2w_ref[...], preferred_element_type=_f32) + f2b_ref[...]
        m = jnp.max(logits, axis=-1, keepdims=True)
        e = jnp.exp(logits - m)
        out_ref[...] = e / jnp.sum(e, axis=-1, keepdims=True)


def _nd(block, im):
    return pl.BlockSpec(block, im)


_full2 = lambda shape: pl.BlockSpec(shape, lambda i: (0, 0))
_rows = lambda w: pl.BlockSpec((_R, w), lambda i: (i, 0))
_parts = pl.BlockSpec((2, _R, _D), lambda i: (0, i, 0))


def _k1_call(degp, x_pad):
    return pl.pallas_call(
        _k1_body,
        grid=(_G,),
        in_specs=[_parts, _rows(_D)],
        out_specs=[_rows(1), _rows(_D), _rows(_D)],
        out_shape=[jax.ShapeDtypeStruct((_NP, 1), _f32),
                   jax.ShapeDtypeStruct((_NP, _D), _f32),
                   jax.ShapeDtypeStruct((_NP, _D), _f32)],
    )(degp, x_pad)


def _k2_call(p, u, dinv):
    return pl.pallas_call(
        _k2_body,
        grid=(_G,),
        in_specs=[_parts, _rows(_D), _rows(1)],
        out_specs=[_rows(_D), _rows(_D)],
        out_shape=[jax.ShapeDtypeStruct((_NP, _D), _f32),
                   jax.ShapeDtypeStruct((_NP, _D), _f32)],
    )(p, u, dinv)


def _k3_call(p, u, dinv, W, b):
    return pl.pallas_call(
        _k3_body,
        grid=(_G,),
        in_specs=[_parts, _rows(_D), _rows(1), _full2((_D, _D)), _full2((1, _D))],
        out_specs=[_rows(_D), _rows(_D)],
        out_shape=[jax.ShapeDtypeStruct((_NP, _D), _f32),
                   jax.ShapeDtypeStruct((_NP, _D), _f32)],
    )(p, u, dinv, W, b)


def _k5_call(p, u, dinv, W2, b2, batch2d, f1w, f1b, f2w, f2b):
    return pl.pallas_call(
        _k5_body,
        grid=(_G,),
        in_specs=[_parts, _rows(_D), _rows(1), _full2((_D, _D)), _full2((1, _D)),
                  _rows(1), _full2((_D, _FC)), _full2((1, _FC)),
                  _full2((_FC, _NC)), _full2((1, _NC))],
        out_specs=_full2((_NG, _NC)),
        out_shape=jax.ShapeDtypeStruct((_NG, _NC), _f32),
        scratch_shapes=[pltpu.VMEM((_NG, _D), _f32),
                        pltpu.VMEM((_NG, _D), _f32)],
    )(p, u, dinv, W2, b2, batch2d, f1w, f1b, f2w, f2b)


def kernel(x, edge_index, batch, W1, b1, W2, b2, fc1_w, fc1_b, fc2_w, fc2_b):
    src = edge_index[0]
    dst = edge_index[1]
    pad_e = jnp.full((_EP - _E,), _N, jnp.int32)
    src2d = jnp.concatenate([src, pad_e]).reshape(_EP // _B, _B)
    dst2d = jnp.concatenate([dst, pad_e]).reshape(_EP // _B, _B)
    x_pad = jnp.concatenate([x, jnp.zeros((_NP - _N, _D), _f32)], axis=0)
    batch2d = jnp.concatenate(
        [batch, jnp.full((_NP - _N,), _NG, jnp.int32)]).reshape(_NP, 1)

    deg = _deg_kernel()
    prop = _prop_kernel()

    degp = deg(dst2d)
    dinv, t0, t0b = _k1_call(degp, x_pad)
    p = prop(t0, t0b, src2d, dst2d)
    t1, t1b = _k2_call(p, t0, dinv)
    p = prop(t1, t1b, src2d, dst2d)
    g2, g2b = _k3_call(p, t1, dinv, W1, b1.reshape(1, _D))
    p = prop(g2, g2b, src2d, dst2d)
    t3, t3b = _k2_call(p, g2, dinv)
    p = prop(t3, t3b, src2d, dst2d)
    out = _k5_call(p, t3, dinv, W2, b2.reshape(1, _D), batch2d,
                   fc1_w, fc1_b.reshape(1, _FC), fc2_w, fc2_b.reshape(1, _NC))
    return out


# speedup vs baseline: 3.3249x; 2.1829x over previous
import functools

import jax
import jax.numpy as jnp
from jax import lax
from jax.experimental import pallas as pl
from jax.experimental.pallas import tpu as pltpu
from jax.experimental.pallas import tpu_sc as plsc

_N = 10000
_E = 320000
_NG = 16
_D = 128
_FC = 128
_NC = 10

_NP = 10240
_EP = 327680
_B = 128
_K = 2
_NW = 32
_NBLK = (_EP // _NW) // _B
_NCHUNK = _NBLK // _K
_RPT = _NP // 16
_R = 1280
_G = _NP // _R

_f32 = jnp.float32


_C = 4
_NCH = _NBLK // _C

_NBLK0 = 60
_NBLK1 = 100
_NCH0 = _NBLK0 // _C
_NCH1 = _NBLK1 // _C


def _prop_body(g_hbm, src_hbm, dst_hbm, out_hbm, sidx, didx, rows, acc,
               isem, gsem, ssem0, ssem1):
    c = lax.axis_index("c")
    s = lax.axis_index("s")
    ssems = (ssem0, ssem1)

    def zrow(i, _):
        for j in range(_D // 16):
            rows[0, i, pl.ds(j * 16, 16)] = jnp.zeros((16,), _f32)
        return 0
    lax.fori_loop(0, _B, zrow, 0)
    rbase = s * _RPT
    for r in range(_RPT // _B):
        pltpu.sync_copy(rows.at[0], acc.at[pl.ds(rbase + r * _B, _B)])
    plsc.subcore_barrier()

    row0 = lax.select(c == 0, s * _NBLK0, 16 * _NBLK0 + s * _NBLK1)
    nch = lax.select(c == 0, _NCH0, _NCH1)

    def ifetch(ci):
        h = lax.rem(ci, 2) * _C
        pltpu.async_copy(src_hbm.at[pl.ds(row0 + ci * _C, _C)],
                         sidx.at[pl.ds(h, _C)], isem)
        pltpu.async_copy(dst_hbm.at[pl.ds(row0 + ci * _C, _C)],
                         didx.at[pl.ds(h, _C)], isem)

    def iwait():
        for _ in range(2):
            pltpu.make_async_copy(src_hbm.at[pl.ds(0, _C)],
                                  sidx.at[pl.ds(0, _C)], isem).wait()

    def swait(p):
        pltpu.make_async_copy(rows.at[0], acc.at[pl.ds(0, _B)],
                              ssems[p]).wait()

    ifetch(0)

    def chunk(ci, _):
        h = lax.rem(ci, 2) * _C
        iwait()
        pltpu.async_copy(g_hbm.at[sidx.at[h]], rows.at[0], gsem)

        @pl.when(ci > 0)
        def _():
            swait(1)
        ifetch(lax.rem(ci + 1, nch))

        for j in range(_C):
            pltpu.make_async_copy(g_hbm.at[sidx.at[h]], rows.at[j % 2],
                                  gsem).wait()
            pltpu.async_copy(rows.at[j % 2], acc.at[didx.at[h + j]],
                             ssems[j % 2], add=True)
            if j > 0:
                swait((j - 1) % 2)
            if j < _C - 1:
                pltpu.async_copy(g_hbm.at[sidx.at[h + j + 1]],
                                 rows.at[(j + 1) % 2], gsem)
        return 0
    lax.fori_loop(0, nch, chunk, 0)

    swait(1)
    iwait()
    plsc.subcore_barrier()
    pltpu.sync_copy(acc.at[pl.ds(rbase, _RPT)],
                    out_hbm.at[c, pl.ds(rbase, _RPT)])


@functools.cache
def _prop_kernel():
    mesh = plsc.VectorSubcoreMesh(core_axis_name="c", subcore_axis_name="s")
    return pl.kernel(
        _prop_body, mesh=mesh,
        out_type=jax.ShapeDtypeStruct((2, _NP, _D), _f32),
        scratch_types=[
            pltpu.VMEM((2 * _C, _B), jnp.int32),
            pltpu.VMEM((2 * _C, _B), jnp.int32),
            pltpu.VMEM((2, _B, _D), _f32),
            pltpu.VMEM_SHARED((_NP, _D), _f32),
            pltpu.SemaphoreType.DMA,
            pltpu.SemaphoreType.DMA,
            pltpu.SemaphoreType.DMA,
            pltpu.SemaphoreType.DMA,
        ],
    )


def _deg_body(dst_hbm, out_hbm, didx, buf, acc, isem, ssem):
    c = lax.axis_index("c")
    s = lax.axis_index("s")
    wid = s * 2 + c

    def zrow(i, _):
        for j in range(_D // 16):
            buf[i, pl.ds(j * 16, 16)] = jnp.zeros((16,), _f32)
        return 0
    lax.fori_loop(0, _B, zrow, 0)
    rbase = s * _RPT
    for r in range(_RPT // _B):
        pltpu.sync_copy(buf, acc.at[pl.ds(rbase + r * _B, _B)])

    def orow(i, _):
        for j in range(_D // 16):
            buf[i, pl.ds(j * 16, 16)] = jnp.ones((16,), _f32)
        return 0
    lax.fori_loop(0, _B, orow, 0)
    plsc.subcore_barrier()

    row0 = wid * _NBLK

    def ifetch(ci):
        h = lax.rem(ci, 2) * _C
        pltpu.async_copy(dst_hbm.at[pl.ds(row0 + ci * _C, _C)],
                         didx.at[pl.ds(h, _C)], isem)

    def iwait():
        pltpu.make_async_copy(dst_hbm.at[pl.ds(0, _C)],
                              didx.at[pl.ds(0, _C)], isem).wait()

    def swait():
        pltpu.make_async_copy(buf, acc.at[pl.ds(0, _B)], ssem).wait()

    ifetch(0)

    def chunk(ci, _):
        h = lax.rem(ci, 2) * _C
        iwait()
        @pl.when(ci > 0)
        def _():
            for _k in range(_C):
                swait()
        ifetch(lax.rem(ci + 1, _NCH))
        for j in range(_C):
            pltpu.async_copy(buf, acc.at[didx.at[h + j]], ssem, add=True)
        return 0
    lax.fori_loop(0, _NCH, chunk, 0)

    for _k in range(_C):
        swait()
    iwait()
    plsc.subcore_barrier()
    pltpu.sync_copy(acc.at[pl.ds(rbase, _RPT)],
                    out_hbm.at[c, pl.ds(rbase, _RPT)])


@functools.cache
def _deg_kernel():
    mesh = plsc.VectorSubcoreMesh(core_axis_name="c", subcore_axis_name="s")
    return pl.kernel(
        _deg_body, mesh=mesh,
        out_type=jax.ShapeDtypeStruct((2, _NP, _D), _f32),
        scratch_types=[
            pltpu.VMEM((2 * _C, _B), jnp.int32),
            pltpu.VMEM((_B, _D), _f32),
            pltpu.VMEM_SHARED((_NP, _D), _f32),
            pltpu.SemaphoreType.DMA,
            pltpu.SemaphoreType.DMA,
        ],
    )


def _k1_body(degp_ref, x_ref, dinv_ref, t0_ref):
    i = pl.program_id(0)
    deg = degp_ref[0][:, 0:1] + degp_ref[1][:, 0:1] + 1.0
    d = lax.rsqrt(deg)
    rid = lax.broadcasted_iota(jnp.int32, (_R, 1), 0) + i * _R
    d = jnp.where(rid < _N, d, 0.0)
    dinv_ref[...] = d
    t0_ref[...] = d * x_ref[...]


def _k2_body(p_ref, u_ref, dinv_ref, o_ref):
    d = dinv_ref[...]
    o_ref[...] = (d * d) * (p_ref[0] + p_ref[1] + u_ref[...])


def _k3_body(p_ref, u_ref, dinv_ref, w_ref, b_ref, o_ref):
    d = dinv_ref[...]
    h = d * (p_ref[0] + p_ref[1] + u_ref[...])
    a = jnp.dot(h, w_ref[...], preferred_element_type=_f32) + b_ref[...]
    o_ref[...] = d * jnp.maximum(a, 0.0)


def _k5_body(p_ref, u_ref, dinv_ref, w2_ref, b2_ref, batch_ref,
             f1w_ref, f1b_ref, f2w_ref, f2b_ref, out_ref, sums, counts):
    i = pl.program_id(0)

    @pl.when(i == 0)
    def _():
        sums[...] = jnp.zeros_like(sums)
        counts[...] = jnp.zeros_like(counts)

    d = dinv_ref[...]
    h = d * (p_ref[0] + p_ref[1] + u_ref[...])
    a = jnp.dot(h, w2_ref[...], preferred_element_type=_f32) + b2_ref[...]
    a = jnp.maximum(a, 0.0)
    onehot = (batch_ref[...] ==
              lax.broadcasted_iota(jnp.int32, (1, _NG), 1)).astype(_f32)
    sums[...] += lax.dot_general(onehot, a, (((0,), (0,)), ((), ())),
                                 preferred_element_type=_f32)
    counts[...] += lax.dot_general(onehot, jnp.ones((_R, _D), _f32),
                                   (((0,), (0,)), ((), ())),
                                   preferred_element_type=_f32)

    @pl.when(i == _G - 1)
    def _():
        g = sums[...] / jnp.maximum(counts[...], 1.0)
        z = jnp.dot(g, f1w_ref[...], preferred_element_type=_f32) + f1b_ref[...]
        z = jnp.maximum(z, 0.0)
        logits = jnp.dot(z, f2w_ref[...], preferred_element_type=_f32) + f2b_ref[...]
        m = jnp.max(logits, axis=-1, keepdims=True)
        e = jnp.exp(logits - m)
        out_ref[...] = e / jnp.sum(e, axis=-1, keepdims=True)


def _nd(block, im):
    return pl.BlockSpec(block, im)


_full2 = lambda shape: pl.BlockSpec(shape, lambda i: (0, 0))
_rows = lambda w: pl.BlockSpec((_R, w), lambda i: (i, 0))
_parts = pl.BlockSpec((2, _R, _D), lambda i: (0, i, 0))


def _k1_call(degp, x_pad):
    return pl.pallas_call(
        _k1_body,
        grid=(_G,),
        in_specs=[_parts, _rows(_D)],
        out_specs=[_rows(1), _rows(_D)],
        out_shape=[jax.ShapeDtypeStruct((_NP, 1), _f32),
                   jax.ShapeDtypeStruct((_NP, _D), _f32)],
    )(degp, x_pad)


def _k2_call(p, u, dinv):
    return pl.pallas_call(
        _k2_body,
        grid=(_G,),
        in_specs=[_parts, _rows(_D), _rows(1)],
        out_specs=_rows(_D),
        out_shape=jax.ShapeDtypeStruct((_NP, _D), _f32),
    )(p, u, dinv)


def _k3_call(p, u, dinv, W, b):
    return pl.pallas_call(
        _k3_body,
        grid=(_G,),
        in_specs=[_parts, _rows(_D), _rows(1), _full2((_D, _D)), _full2((1, _D))],
        out_specs=_rows(_D),
        out_shape=jax.ShapeDtypeStruct((_NP, _D), _f32),
    )(p, u, dinv, W, b)


def _k5_call(p, u, dinv, W2, b2, batch2d, f1w, f1b, f2w, f2b):
    return pl.pallas_call(
        _k5_body,
        grid=(_G,),
        in_specs=[_parts, _rows(_D), _rows(1), _full2((_D, _D)), _full2((1, _D)),
                  _rows(1), _full2((_D, _FC)), _full2((1, _FC)),
                  _full2((_FC, _NC)), _full2((1, _NC))],
        out_specs=_full2((_NG, _NC)),
        out_shape=jax.ShapeDtypeStruct((_NG, _NC), _f32),
        scratch_shapes=[pltpu.VMEM((_NG, _D), _f32),
                        pltpu.VMEM((_NG, _D), _f32)],
    )(p, u, dinv, W2, b2, batch2d, f1w, f1b, f2w, f2b)


def kernel(x, edge_index, batch, W1, b1, W2, b2, fc1_w, fc1_b, fc2_w, fc2_b):
    src = edge_index[0]
    dst = edge_index[1]
    pad_e = _N + (jnp.arange(_EP - _E, dtype=jnp.int32) % (_NP - _N))
    src2d = jnp.concatenate([src, pad_e]).reshape(_EP // _B, _B)
    dst2d = jnp.concatenate([dst, pad_e]).reshape(_EP // _B, _B)
    x_pad = jnp.concatenate([x, jnp.zeros((_NP - _N, _D), _f32)], axis=0)
    batch2d = jnp.concatenate(
        [batch, jnp.full((_NP - _N,), _NG, jnp.int32)]).reshape(_NP, 1)

    deg = _deg_kernel()
    prop = _prop_kernel()

    degp = deg(dst2d)
    dinv, t0 = _k1_call(degp, x_pad)
    p = prop(t0, src2d, dst2d)
    t1 = _k2_call(p, t0, dinv)
    p = prop(t1, src2d, dst2d)
    g2 = _k3_call(p, t1, dinv, W1, b1.reshape(1, _D))
    p = prop(g2, src2d, dst2d)
    t3 = _k2_call(p, g2, dinv)
    p = prop(t3, src2d, dst2d)
    out = _k5_call(p, t3, dinv, W2, b2.reshape(1, _D), batch2d,
                   fc1_w, fc1_b.reshape(1, _FC), fc2_w, fc2_b.reshape(1, _NC))
    return out


# speedup vs baseline: 3.6006x; 1.0829x over previous
import functools

import jax
import jax.numpy as jnp
from jax import lax
from jax.experimental import pallas as pl
from jax.experimental.pallas import tpu as pltpu
from jax.experimental.pallas import tpu_sc as plsc

_N = 10000
_E = 320000
_NG = 16
_D = 128
_FC = 128
_NC = 10

_NP = 10240
_EP = 327680
_B = 128
_K = 2
_NW = 32
_NBLK = (_EP // _NW) // _B
_NCHUNK = _NBLK // _K
_RPT = _NP // 16
_R = 1280
_G = _NP // _R

_f32 = jnp.float32


_C = 4
_NCH = _NBLK // _C

_NBLK0 = 88
_NBLK1 = 72
_NCH0 = _NBLK0 // _C
_NCH1 = _NBLK1 // _C


def _prop_body(g_hbm, src_hbm, dst_hbm, out_hbm, sidx, didx, rows, acc,
               isem, gsem, ssem0, ssem1):
    c = lax.axis_index("c")
    s = lax.axis_index("s")
    ssems = (ssem0, ssem1)

    def zrow(i, _):
        for j in range(_D // 16):
            rows[0, i, pl.ds(j * 16, 16)] = jnp.zeros((16,), _f32)
        return 0
    lax.fori_loop(0, _B, zrow, 0)
    rbase = s * _RPT
    for r in range(_RPT // _B):
        pltpu.sync_copy(rows.at[0], acc.at[pl.ds(rbase + r * _B, _B)])
    plsc.subcore_barrier()

    row0 = lax.select(c == 0, s * _NBLK0, 16 * _NBLK0 + s * _NBLK1)
    nch = lax.select(c == 0, _NCH0, _NCH1)

    def ifetch(ci):
        h = lax.rem(ci, 2) * _C
        pltpu.async_copy(src_hbm.at[pl.ds(row0 + ci * _C, _C)],
                         sidx.at[pl.ds(h, _C)], isem)
        pltpu.async_copy(dst_hbm.at[pl.ds(row0 + ci * _C, _C)],
                         didx.at[pl.ds(h, _C)], isem)

    def iwait():
        for _ in range(2):
            pltpu.make_async_copy(src_hbm.at[pl.ds(0, _C)],
                                  sidx.at[pl.ds(0, _C)], isem).wait()

    def swait(p):
        pltpu.make_async_copy(rows.at[0], acc.at[pl.ds(0, _B)],
                              ssems[p]).wait()

    ifetch(0)

    def chunk(ci, _):
        h = lax.rem(ci, 2) * _C
        iwait()
        pltpu.async_copy(g_hbm.at[sidx.at[h]], rows.at[0], gsem)

        @pl.when(ci > 0)
        def _():
            swait(1)
        ifetch(lax.rem(ci + 1, nch))

        for j in range(_C):
            pltpu.make_async_copy(g_hbm.at[sidx.at[h]], rows.at[j % 2],
                                  gsem).wait()
            pltpu.async_copy(rows.at[j % 2], acc.at[didx.at[h + j]],
                             ssems[j % 2], add=True)
            if j > 0:
                swait((j - 1) % 2)
            if j < _C - 1:
                pltpu.async_copy(g_hbm.at[sidx.at[h + j + 1]],
                                 rows.at[(j + 1) % 2], gsem)
        return 0
    lax.fori_loop(0, nch, chunk, 0)

    swait(1)
    iwait()
    plsc.subcore_barrier()
    pltpu.sync_copy(acc.at[pl.ds(rbase, _RPT)],
                    out_hbm.at[c, pl.ds(rbase, _RPT)])


@functools.cache
def _prop_kernel():
    mesh = plsc.VectorSubcoreMesh(core_axis_name="c", subcore_axis_name="s")
    return pl.kernel(
        _prop_body, mesh=mesh,
        out_type=jax.ShapeDtypeStruct((2, _NP, _D), _f32),
        scratch_types=[
            pltpu.VMEM((2 * _C, _B), jnp.int32),
            pltpu.VMEM((2 * _C, _B), jnp.int32),
            pltpu.VMEM((2, _B, _D), _f32),
            pltpu.VMEM_SHARED((_NP, _D), _f32),
            pltpu.SemaphoreType.DMA,
            pltpu.SemaphoreType.DMA,
            pltpu.SemaphoreType.DMA,
            pltpu.SemaphoreType.DMA,
        ],
    )


def _deg_body(dst_hbm, out_hbm, didx, buf, acc, isem, ssem):
    c = lax.axis_index("c")
    s = lax.axis_index("s")
    wid = s * 2 + c

    def zrow(i, _):
        for j in range(_D // 16):
            buf[i, pl.ds(j * 16, 16)] = jnp.zeros((16,), _f32)
        return 0
    lax.fori_loop(0, _B, zrow, 0)
    rbase = s * _RPT
    for r in range(_RPT // _B):
        pltpu.sync_copy(buf, acc.at[pl.ds(rbase + r * _B, _B)])

    def orow(i, _):
        for j in range(_D // 16):
            buf[i, pl.ds(j * 16, 16)] = jnp.ones((16,), _f32)
        return 0
    lax.fori_loop(0, _B, orow, 0)
    plsc.subcore_barrier()

    row0 = wid * _NBLK

    def ifetch(ci):
        h = lax.rem(ci, 2) * _C
        pltpu.async_copy(dst_hbm.at[pl.ds(row0 + ci * _C, _C)],
                         didx.at[pl.ds(h, _C)], isem)

    def iwait():
        pltpu.make_async_copy(dst_hbm.at[pl.ds(0, _C)],
                              didx.at[pl.ds(0, _C)], isem).wait()

    def swait():
        pltpu.make_async_copy(buf, acc.at[pl.ds(0, _B)], ssem).wait()

    ifetch(0)

    def chunk(ci, _):
        h = lax.rem(ci, 2) * _C
        iwait()
        @pl.when(ci > 0)
        def _():
            for _k in range(_C):
                swait()
        ifetch(lax.rem(ci + 1, _NCH))
        for j in range(_C):
            pltpu.async_copy(buf, acc.at[didx.at[h + j]], ssem, add=True)
        return 0
    lax.fori_loop(0, _NCH, chunk, 0)

    for _k in range(_C):
        swait()
    iwait()
    plsc.subcore_barrier()
    pltpu.sync_copy(acc.at[pl.ds(rbase, _RPT)],
                    out_hbm.at[c, pl.ds(rbase, _RPT)])


@functools.cache
def _deg_kernel():
    mesh = plsc.VectorSubcoreMesh(core_axis_name="c", subcore_axis_name="s")
    return pl.kernel(
        _deg_body, mesh=mesh,
        out_type=jax.ShapeDtypeStruct((2, _NP, _D), _f32),
        scratch_types=[
            pltpu.VMEM((2 * _C, _B), jnp.int32),
            pltpu.VMEM((_B, _D), _f32),
            pltpu.VMEM_SHARED((_NP, _D), _f32),
            pltpu.SemaphoreType.DMA,
            pltpu.SemaphoreType.DMA,
        ],
    )


def _k1_body(degp_ref, x_ref, dinv_ref, t0_ref):
    i = pl.program_id(0)
    deg = degp_ref[0][:, 0:1] + degp_ref[1][:, 0:1] + 1.0
    d = lax.rsqrt(deg)
    rid = lax.broadcasted_iota(jnp.int32, (_R, 1), 0) + i * _R
    d = jnp.where(rid < _N, d, 0.0)
    dinv_ref[...] = d
    t0_ref[...] = d * x_ref[...]


def _k2_body(p_ref, u_ref, dinv_ref, o_ref):
    d = dinv_ref[...]
    o_ref[...] = (d * d) * (p_ref[0] + p_ref[1] + u_ref[...])


def _k3_body(p_ref, u_ref, dinv_ref, w_ref, b_ref, o_ref):
    d = dinv_ref[...]
    h = d * (p_ref[0] + p_ref[1] + u_ref[...])
    a = jnp.dot(h, w_ref[...], preferred_element_type=_f32) + b_ref[...]
    o_ref[...] = d * jnp.maximum(a, 0.0)


def _k5_body(p_ref, u_ref, dinv_ref, w2_ref, b2_ref, batch_ref,
             f1w_ref, f1b_ref, f2w_ref, f2b_ref, out_ref, sums, counts):
    i = pl.program_id(0)

    @pl.when(i == 0)
    def _():
        sums[...] = jnp.zeros_like(sums)
        counts[...] = jnp.zeros_like(counts)

    d = dinv_ref[...]
    h = d * (p_ref[0] + p_ref[1] + u_ref[...])
    a = jnp.dot(h, w2_ref[...], preferred_element_type=_f32) + b2_ref[...]
    a = jnp.maximum(a, 0.0)
    onehot = (batch_ref[...] ==
              lax.broadcasted_iota(jnp.int32, (1, _NG), 1)).astype(_f32)
    sums[...] += lax.dot_general(onehot, a, (((0,), (0,)), ((), ())),
                                 preferred_element_type=_f32)
    counts[...] += lax.dot_general(onehot, jnp.ones((_R, _D), _f32),
                                   (((0,), (0,)), ((), ())),
                                   preferred_element_type=_f32)

    @pl.when(i == _G - 1)
    def _():
        g = sums[...] / jnp.maximum(counts[...], 1.0)
        z = jnp.dot(g, f1w_ref[...], preferred_element_type=_f32) + f1b_ref[...]
        z = jnp.maximum(z, 0.0)
        logits = jnp.dot(z, f2w_ref[...], preferred_element_type=_f32) + f2b_ref[...]
        m = jnp.max(logits, axis=-1, keepdims=True)
        e = jnp.exp(logits - m)
        out_ref[...] = e / jnp.sum(e, axis=-1, keepdims=True)


def _nd(block, im):
    return pl.BlockSpec(block, im)


_full2 = lambda shape: pl.BlockSpec(shape, lambda i: (0, 0))
_rows = lambda w: pl.BlockSpec((_R, w), lambda i: (i, 0))
_parts = pl.BlockSpec((2, _R, _D), lambda i: (0, i, 0))


def _k1_call(degp, x_pad):
    return pl.pallas_call(
        _k1_body,
        grid=(_G,),
        in_specs=[_parts, _rows(_D)],
        out_specs=[_rows(1), _rows(_D)],
        out_shape=[jax.ShapeDtypeStruct((_NP, 1), _f32),
                   jax.ShapeDtypeStruct((_NP, _D), _f32)],
    )(degp, x_pad)


def _k2_call(p, u, dinv):
    return pl.pallas_call(
        _k2_body,
        grid=(_G,),
        in_specs=[_parts, _rows(_D), _rows(1)],
        out_specs=_rows(_D),
        out_shape=jax.ShapeDtypeStruct((_NP, _D), _f32),
    )(p, u, dinv)


def _k3_call(p, u, dinv, W, b):
    return pl.pallas_call(
        _k3_body,
        grid=(_G,),
        in_specs=[_parts, _rows(_D), _rows(1), _full2((_D, _D)), _full2((1, _D))],
        out_specs=_rows(_D),
        out_shape=jax.ShapeDtypeStruct((_NP, _D), _f32),
    )(p, u, dinv, W, b)


def _k5_call(p, u, dinv, W2, b2, batch2d, f1w, f1b, f2w, f2b):
    return pl.pallas_call(
        _k5_body,
        grid=(_G,),
        in_specs=[_parts, _rows(_D), _rows(1), _full2((_D, _D)), _full2((1, _D)),
                  _rows(1), _full2((_D, _FC)), _full2((1, _FC)),
                  _full2((_FC, _NC)), _full2((1, _NC))],
        out_specs=_full2((_NG, _NC)),
        out_shape=jax.ShapeDtypeStruct((_NG, _NC), _f32),
        scratch_shapes=[pltpu.VMEM((_NG, _D), _f32),
                        pltpu.VMEM((_NG, _D), _f32)],
    )(p, u, dinv, W2, b2, batch2d, f1w, f1b, f2w, f2b)


def kernel(x, edge_index, batch, W1, b1, W2, b2, fc1_w, fc1_b, fc2_w, fc2_b):
    src = edge_index[0]
    dst = edge_index[1]
    pad_e = _N + (jnp.arange(_EP - _E, dtype=jnp.int32) % (_NP - _N))
    src2d = jnp.concatenate([src, pad_e]).reshape(_EP // _B, _B)
    dst2d = jnp.concatenate([dst, pad_e]).reshape(_EP // _B, _B)
    x_pad = jnp.concatenate([x, jnp.zeros((_NP - _N, _D), _f32)], axis=0)
    batch2d = jnp.concatenate(
        [batch, jnp.full((_NP - _N,), _NG, jnp.int32)]).reshape(_NP, 1)

    deg = _deg_kernel()
    prop = _prop_kernel()

    degp = deg(dst2d)
    dinv, t0 = _k1_call(degp, x_pad)
    p = prop(t0, src2d, dst2d)
    t1 = _k2_call(p, t0, dinv)
    p = prop(t1, src2d, dst2d)
    g2 = _k3_call(p, t1, dinv, W1, b1.reshape(1, _D))
    p = prop(g2, src2d, dst2d)
    t3 = _k2_call(p, g2, dinv)
    p = prop(t3, src2d, dst2d)
    out = _k5_call(p, t3, dinv, W2, b2.reshape(1, _D), batch2d,
                   fc1_w, fc1_b.reshape(1, _FC), fc2_w, fc2_b.reshape(1, _NC))
    return out


# speedup vs baseline: 3.8422x; 1.0671x over previous
import functools

import jax
import jax.numpy as jnp
from jax import lax
from jax.experimental import pallas as pl
from jax.experimental.pallas import tpu as pltpu
from jax.experimental.pallas import tpu_sc as plsc

_N = 10000
_E = 320000
_NG = 16
_D = 128
_FC = 128
_NC = 10

_NP = 10240
_EP = 327680
_B = 128
_K = 2
_NW = 32
_NBLK = (_EP // _NW) // _B
_NCHUNK = _NBLK // _K
_RPT = _NP // 16
_R = 1280
_G = _NP // _R

_f32 = jnp.float32


_C = 4
_NCH = _NBLK // _C

_NBLK0 = 80
_NBLK1 = 80
_NCH0 = _NBLK0 // _C
_NCH1 = _NBLK1 // _C


def _prop_body(g_hbm, src_hbm, dst_hbm, out_hbm, sidx, didx, rows, acc,
               isem, gsem, ssem0, ssem1):
    c = lax.axis_index("c")
    s = lax.axis_index("s")
    ssems = (ssem0, ssem1)

    def zrow(i, _):
        for j in range(_D // 16):
            rows[0, i, pl.ds(j * 16, 16)] = jnp.zeros((16,), _f32)
        return 0
    lax.fori_loop(0, _B, zrow, 0)
    rbase = s * _RPT
    for r in range(_RPT // _B):
        pltpu.sync_copy(rows.at[0], acc.at[pl.ds(rbase + r * _B, _B)])
    plsc.subcore_barrier()

    row0 = lax.select(c == 0, s * _NBLK0, 16 * _NBLK0 + s * _NBLK1)
    nch = lax.select(c == 0, _NCH0, _NCH1)

    def ifetch(ci):
        h = lax.rem(ci, 2) * _C
        pltpu.async_copy(src_hbm.at[pl.ds(row0 + ci * _C, _C)],
                         sidx.at[pl.ds(h, _C)], isem)
        pltpu.async_copy(dst_hbm.at[pl.ds(row0 + ci * _C, _C)],
                         didx.at[pl.ds(h, _C)], isem)

    def iwait():
        for _ in range(2):
            pltpu.make_async_copy(src_hbm.at[pl.ds(0, _C)],
                                  sidx.at[pl.ds(0, _C)], isem).wait()

    def swait(p):
        pltpu.make_async_copy(rows.at[0], acc.at[pl.ds(0, _B)],
                              ssems[p]).wait()

    ifetch(0)

    def chunk(ci, _):
        h = lax.rem(ci, 2) * _C
        iwait()
        pltpu.async_copy(g_hbm.at[sidx.at[h]], rows.at[0], gsem)

        @pl.when(ci > 0)
        def _():
            swait(1)
        ifetch(lax.rem(ci + 1, nch))

        for j in range(_C):
            pltpu.make_async_copy(g_hbm.at[sidx.at[h]], rows.at[j % 2],
                                  gsem).wait()
            pltpu.async_copy(rows.at[j % 2], acc.at[didx.at[h + j]],
                             ssems[j % 2], add=True)
            if j > 0:
                swait((j - 1) % 2)
            if j < _C - 1:
                pltpu.async_copy(g_hbm.at[sidx.at[h + j + 1]],
                                 rows.at[(j + 1) % 2], gsem)
        return 0
    lax.fori_loop(0, nch, chunk, 0)

    swait(1)
    iwait()
    plsc.subcore_barrier()
    pltpu.sync_copy(acc.at[pl.ds(rbase, _RPT)],
                    out_hbm.at[c, pl.ds(rbase, _RPT)])


@functools.cache
def _prop_kernel():
    mesh = plsc.VectorSubcoreMesh(core_axis_name="c", subcore_axis_name="s")
    return pl.kernel(
        _prop_body, mesh=mesh,
        out_type=jax.ShapeDtypeStruct((2, _NP, _D), _f32),
        scratch_types=[
            pltpu.VMEM((2 * _C, _B), jnp.int32),
            pltpu.VMEM((2 * _C, _B), jnp.int32),
            pltpu.VMEM((2, _B, _D), _f32),
            pltpu.VMEM_SHARED((_NP, _D), _f32),
            pltpu.SemaphoreType.DMA,
            pltpu.SemaphoreType.DMA,
            pltpu.SemaphoreType.DMA,
            pltpu.SemaphoreType.DMA,
        ],
    )


def _deg_body(dst_hbm, out_hbm, didx, buf, acc, isem, ssem):
    c = lax.axis_index("c")
    s = lax.axis_index("s")
    wid = s * 2 + c

    def zrow(i, _):
        for j in range(_D // 16):
            buf[i, pl.ds(j * 16, 16)] = jnp.zeros((16,), _f32)
        return 0
    lax.fori_loop(0, _B, zrow, 0)
    rbase = s * _RPT
    for r in range(_RPT // _B):
        pltpu.sync_copy(buf, acc.at[pl.ds(rbase + r * _B, _B)])

    def orow(i, _):
        for j in range(_D // 16):
            buf[i, pl.ds(j * 16, 16)] = jnp.ones((16,), _f32)
        return 0
    lax.fori_loop(0, _B, orow, 0)
    plsc.subcore_barrier()

    row0 = wid * _NBLK

    def ifetch(ci):
        h = lax.rem(ci, 2) * _C
        pltpu.async_copy(dst_hbm.at[pl.ds(row0 + ci * _C, _C)],
                         didx.at[pl.ds(h, _C)], isem)

    def iwait():
        pltpu.make_async_copy(dst_hbm.at[pl.ds(0, _C)],
                              didx.at[pl.ds(0, _C)], isem).wait()

    def swait():
        pltpu.make_async_copy(buf, acc.at[pl.ds(0, _B)], ssem).wait()

    ifetch(0)

    def chunk(ci, _):
        h = lax.rem(ci, 2) * _C
        iwait()
        @pl.when(ci > 0)
        def _():
            for _k in range(_C):
                swait()
        ifetch(lax.rem(ci + 1, _NCH))
        for j in range(_C):
            pltpu.async_copy(buf, acc.at[didx.at[h + j]], ssem, add=True)
        return 0
    lax.fori_loop(0, _NCH, chunk, 0)

    for _k in range(_C):
        swait()
    iwait()
    plsc.subcore_barrier()
    pltpu.sync_copy(acc.at[pl.ds(rbase, _RPT)],
                    out_hbm.at[c, pl.ds(rbase, _RPT)])


@functools.cache
def _deg_kernel():
    mesh = plsc.VectorSubcoreMesh(core_axis_name="c", subcore_axis_name="s")
    return pl.kernel(
        _deg_body, mesh=mesh,
        out_type=jax.ShapeDtypeStruct((2, _NP, _D), _f32),
        scratch_types=[
            pltpu.VMEM((2 * _C, _B), jnp.int32),
            pltpu.VMEM((_B, _D), _f32),
            pltpu.VMEM_SHARED((_NP, _D), _f32),
            pltpu.SemaphoreType.DMA,
            pltpu.SemaphoreType.DMA,
        ],
    )


def _k1_body(degp_ref, x_ref, dinv_ref, t0_ref):
    i = pl.program_id(0)
    deg = degp_ref[0][:, 0:1] + degp_ref[1][:, 0:1] + 1.0
    d = lax.rsqrt(deg)
    rid = lax.broadcasted_iota(jnp.int32, (_R, 1), 0) + i * _R
    d = jnp.where(rid < _N, d, 0.0)
    dinv_ref[...] = d
    t0_ref[...] = d * x_ref[...]


def _k2_body(p_ref, u_ref, dinv_ref, o_ref):
    d = dinv_ref[...]
    o_ref[...] = (d * d) * (p_ref[0] + p_ref[1] + u_ref[...])


def _k3_body(p_ref, u_ref, dinv_ref, w_ref, b_ref, o_ref):
    d = dinv_ref[...]
    h = d * (p_ref[0] + p_ref[1] + u_ref[...])
    a = jnp.dot(h, w_ref[...], preferred_element_type=_f32) + b_ref[...]
    o_ref[...] = d * jnp.maximum(a, 0.0)


def _k5_body(p_ref, u_ref, dinv_ref, w2_ref, b2_ref, batch_ref,
             f1w_ref, f1b_ref, f2w_ref, f2b_ref, out_ref, sums, counts):
    i = pl.program_id(0)

    @pl.when(i == 0)
    def _():
        sums[...] = jnp.zeros_like(sums)
        counts[...] = jnp.zeros_like(counts)

    d = dinv_ref[...]
    h = d * (p_ref[0] + p_ref[1] + u_ref[...])
    a = jnp.dot(h, w2_ref[...], preferred_element_type=_f32) + b2_ref[...]
    a = jnp.maximum(a, 0.0)
    onehot = (batch_ref[...] ==
              lax.broadcasted_iota(jnp.int32, (1, _NG), 1)).astype(_f32)
    sums[...] += lax.dot_general(onehot, a, (((0,), (0,)), ((), ())),
                                 preferred_element_type=_f32)
    counts[...] += lax.dot_general(onehot, jnp.ones((_R, _D), _f32),
                                   (((0,), (0,)), ((), ())),
                                   preferred_element_type=_f32)

    @pl.when(i == _G - 1)
    def _():
        g = sums[...] / jnp.maximum(counts[...], 1.0)
        z = jnp.dot(g, f1w_ref[...], preferred_element_type=_f32) + f1b_ref[...]
        z = jnp.maximum(z, 0.0)
        logits = jnp.dot(z, f2w_ref[...], preferred_element_type=_f32) + f2b_ref[...]
        m = jnp.max(logits, axis=-1, keepdims=True)
        e = jnp.exp(logits - m)
        out_ref[...] = e / jnp.sum(e, axis=-1, keepdims=True)


def _nd(block, im):
    return pl.BlockSpec(block, im)


_full2 = lambda shape: pl.BlockSpec(shape, lambda i: (0, 0))
_rows = lambda w: pl.BlockSpec((_R, w), lambda i: (i, 0))
_parts = pl.BlockSpec((2, _R, _D), lambda i: (0, i, 0))


def _k1_call(degp, x_pad):
    return pl.pallas_call(
        _k1_body,
        grid=(_G,),
        in_specs=[_parts, _rows(_D)],
        out_specs=[_rows(1), _rows(_D)],
        out_shape=[jax.ShapeDtypeStruct((_NP, 1), _f32),
                   jax.ShapeDtypeStruct((_NP, _D), _f32)],
    )(degp, x_pad)


def _k2_call(p, u, dinv):
    return pl.pallas_call(
        _k2_body,
        grid=(_G,),
        in_specs=[_parts, _rows(_D), _rows(1)],
        out_specs=_rows(_D),
        out_shape=jax.ShapeDtypeStruct((_NP, _D), _f32),
    )(p, u, dinv)


def _k3_call(p, u, dinv, W, b):
    return pl.pallas_call(
        _k3_body,
        grid=(_G,),
        in_specs=[_parts, _rows(_D), _rows(1), _full2((_D, _D)), _full2((1, _D))],
        out_specs=_rows(_D),
        out_shape=jax.ShapeDtypeStruct((_NP, _D), _f32),
    )(p, u, dinv, W, b)


def _k5_call(p, u, dinv, W2, b2, batch2d, f1w, f1b, f2w, f2b):
    return pl.pallas_call(
        _k5_body,
        grid=(_G,),
        in_specs=[_parts, _rows(_D), _rows(1), _full2((_D, _D)), _full2((1, _D)),
                  _rows(1), _full2((_D, _FC)), _full2((1, _FC)),
                  _full2((_FC, _NC)), _full2((1, _NC))],
        out_specs=_full2((_NG, _NC)),
        out_shape=jax.ShapeDtypeStruct((_NG, _NC), _f32),
        scratch_shapes=[pltpu.VMEM((_NG, _D), _f32),
                        pltpu.VMEM((_NG, _D), _f32)],
    )(p, u, dinv, W2, b2, batch2d, f1w, f1b, f2w, f2b)


def kernel(x, edge_index, batch, W1, b1, W2, b2, fc1_w, fc1_b, fc2_w, fc2_b):
    src = edge_index[0]
    dst = edge_index[1]
    pad_e = _N + (jnp.arange(_EP - _E, dtype=jnp.int32) % (_NP - _N))
    src2d = jnp.concatenate([src, pad_e]).reshape(_EP // _B, _B)
    dst2d = jnp.concatenate([dst, pad_e]).reshape(_EP // _B, _B)
    x_pad = jnp.concatenate([x, jnp.zeros((_NP - _N, _D), _f32)], axis=0)
    batch2d = jnp.concatenate(
        [batch, jnp.full((_NP - _N,), _NG, jnp.int32)]).reshape(_NP, 1)

    deg = _deg_kernel()
    prop = _prop_kernel()

    degp = deg(dst2d)
    dinv, t0 = _k1_call(degp, x_pad)
    p = prop(t0, src2d, dst2d)
    t1 = _k2_call(p, t0, dinv)
    p = prop(t1, src2d, dst2d)
    g2 = _k3_call(p, t1, dinv, W1, b1.reshape(1, _D))
    p = prop(g2, src2d, dst2d)
    t3 = _k2_call(p, g2, dinv)
    p = prop(t3, src2d, dst2d)
    out = _k5_call(p, t3, dinv, W2, b2.reshape(1, _D), batch2d,
                   fc1_w, fc1_b.reshape(1, _FC), fc2_w, fc2_b.reshape(1, _NC))
    return out
